# baseline (speedup 1.0000x reference)
import functools

import jax
import jax.numpy as jnp
from jax import lax
from jax.experimental import pallas as pl
from jax.experimental.pallas import tpu as pltpu
from jax.experimental.pallas import tpu_sc as plsc

_CHUNK = 128


def _gather_call(B, V, D):
  info = plsc.get_sparse_core_info()
  NC, NS = info.num_cores, info.num_subcores
  NW = NC * NS
  assert B % (NW * _CHUNK) == 0
  rows_per_w = B // (NW * _CHUNK)
  b_per_w = B // NW

  mesh = plsc.VectorSubcoreMesh(core_axis_name="c", subcore_axis_name="s")

  @functools.partial(
      pl.kernel,
      mesh=mesh,
      compiler_params=pltpu.CompilerParams(use_tc_tiling_on_sc=False),
      out_type=jax.ShapeDtypeStruct((B, D), jnp.float32),
      scratch_types=[
          pltpu.VMEM((rows_per_w, _CHUNK), jnp.int32),
          pltpu.VMEM((b_per_w, D), jnp.float32),
          pltpu.SemaphoreType.DMA,
      ],
  )
  def gather_kernel(ids_hbm, data_hbm, out_hbm, idx_v, rows_v, sem):
    wid = lax.axis_index("s") * NC + lax.axis_index("c")
    r0 = wid * rows_per_w
    pltpu.sync_copy(ids_hbm.at[pl.ds(r0, rows_per_w)], idx_v)
    copies = [
        pltpu.async_copy(
            data_hbm.at[idx_v.at[j]],
            rows_v.at[pl.ds(j * _CHUNK, _CHUNK)],
            sem,
        )
        for j in range(rows_per_w)
    ]
    for c in copies:
      c.wait()
    pltpu.sync_copy(rows_v, out_hbm.at[pl.ds(wid * b_per_w, b_per_w)])

  return gather_kernel


def kernel(ids, data):
  B, = ids.shape
  V, D = data.shape
  Dp = 8
  data8 = jnp.pad(data, ((0, 0), (0, Dp - D)))
  ids2d = ids.reshape(B // _CHUNK, _CHUNK)
  out = _gather_call(B, V, Dp)(ids2d, data8)
  return out[:, :D]


# speedup vs baseline: 1.1366x; 1.1366x over previous
import functools

import jax
import jax.numpy as jnp
from jax import lax
from jax.experimental import pallas as pl
from jax.experimental.pallas import tpu as pltpu
from jax.experimental.pallas import tpu_sc as plsc

_CHUNK = 128


def _gather_call(B, V, D):
  info = plsc.get_sparse_core_info()
  NC, NS, L = info.num_cores, info.num_subcores, info.num_lanes
  NW = NC * NS
  assert B % (NW * _CHUNK) == 0 and (B // NW * D) % _CHUNK == 0
  b_per_w = B // NW
  words_per_w = b_per_w * D
  idx_rows = words_per_w // _CHUNK

  mesh = plsc.VectorSubcoreMesh(core_axis_name="c", subcore_axis_name="s")

  @functools.partial(
      pl.kernel,
      mesh=mesh,
      compiler_params=pltpu.CompilerParams(
          use_tc_tiling_on_sc=False, needs_layout_passes=False
      ),
      out_type=jax.ShapeDtypeStruct((B * D,), jnp.float32),
      scratch_types=[
          pltpu.VMEM((b_per_w,), jnp.int32),
          pltpu.VMEM((words_per_w,), jnp.int32),
          pltpu.VMEM((words_per_w,), jnp.float32),
          pltpu.SemaphoreType.DMA,
      ],
  )
  def gather_kernel(ids_hbm, data_hbm, out_hbm, ids_v, widx_v, out_v, sem):
    wid = lax.axis_index("s") * NC + lax.axis_index("c")
    pltpu.sync_copy(ids_hbm.at[pl.ds(wid * b_per_w, b_per_w)], ids_v)
    lane = lax.iota(jnp.int32, L)
    for t in range(words_per_w // L):
      q = t * L + lane
      iq = lax.div(q, D)
      rq = q - iq * D
      idv = plsc.load_gather(ids_v, [iq])
      widx_v[pl.ds(t * L, L)] = idv * D + rq
    copies = [
        pltpu.async_copy(
            data_hbm.at[widx_v.at[pl.ds(j * _CHUNK, _CHUNK)]],
            out_v.at[pl.ds(j * _CHUNK, _CHUNK)],
            sem,
        )
        for j in range(idx_rows)
    ]
    for cp in copies:
      cp.wait()
    pltpu.sync_copy(out_v, out_hbm.at[pl.ds(wid * words_per_w, words_per_w)])

  return gather_kernel


def kernel(ids, data):
  B, = ids.shape
  V, D = data.shape
  flat = _gather_call(B, V, D)(ids, data.reshape(V * D))
  return flat.reshape(B, D)


# speedup vs baseline: 2.7267x; 2.3990x over previous
import functools

import jax
import jax.numpy as jnp
from jax import lax
from jax.experimental import pallas as pl
from jax.experimental.pallas import tpu as pltpu
from jax.experimental.pallas import tpu_sc as plsc

_CHUNK = 128


def _gather_call(B, V, D):
  info = plsc.get_sparse_core_info()
  NC, NS, L = info.num_cores, info.num_subcores, info.num_lanes
  NW = NC * NS
  assert B % (NW * _CHUNK) == 0 and (B // NW * D) % _CHUNK == 0
  b_per_w = B // NW
  words_per_w = b_per_w * D
  idx_rows = words_per_w // _CHUNK

  mesh = plsc.VectorSubcoreMesh(core_axis_name="c", subcore_axis_name="s")

  @functools.partial(
      pl.kernel,
      mesh=mesh,
      compiler_params=pltpu.CompilerParams(
          use_tc_tiling_on_sc=False, needs_layout_passes=False
      ),
      out_type=jax.ShapeDtypeStruct((B * D,), jnp.float32),
      scratch_types=[
          pltpu.VMEM((b_per_w,), jnp.int32),
          pltpu.VMEM((words_per_w,), jnp.int32),
          pltpu.VMEM((words_per_w,), jnp.float32),
          pltpu.SemaphoreType.DMA,
      ],
  )
  def gather_kernel(ids_hbm, data_hbm, out_hbm, ids_v, widx_v, out_v, sem):
    wid = lax.axis_index("s") * NC + lax.axis_index("c")
    pltpu.sync_copy(ids_hbm.at[pl.ds(wid * b_per_w, b_per_w)], ids_v)
    lane = lax.iota(jnp.int32, L)
    for t in range(words_per_w // L):
      q = t * L + lane
      iq = lax.div(q, D)
      rq = q - iq * D
      idv = plsc.load_gather(ids_v, [iq])
      widx_v[pl.ds(t * L, L)] = idv + rq * V
    copies = [
        pltpu.async_copy(
            data_hbm.at[widx_v.at[pl.ds(j * _CHUNK, _CHUNK)]],
            out_v.at[pl.ds(j * _CHUNK, _CHUNK)],
            sem,
        )
        for j in range(idx_rows)
    ]
    for cp in copies:
      cp.wait()
    pltpu.sync_copy(out_v, out_hbm.at[pl.ds(wid * words_per_w, words_per_w)])

  return gather_kernel


def kernel(ids, data):
  B, = ids.shape
  V, D = data.shape
  flat = _gather_call(B, V, D)(ids, data.T.reshape(V * D))
  return flat.reshape(B, D)


# speedup vs baseline: 2.8091x; 1.0302x over previous
import functools

import numpy as np
import jax
import jax.numpy as jnp
from jax import lax
from jax.experimental import pallas as pl
from jax.experimental.pallas import tpu as pltpu
from jax.experimental.pallas import tpu_sc as plsc

_CHUNK = 128


def _gather_call(B, V, D):
  info = plsc.get_sparse_core_info()
  NC, NS, L = info.num_cores, info.num_subcores, info.num_lanes
  NW = NC * NS
  assert B % (NW * _CHUNK) == 0 and (B // NW * D) % _CHUNK == 0
  b_per_w = B // NW
  words_per_w = b_per_w * D
  idx_rows = words_per_w // _CHUNK
  vecs_per_row = _CHUNK // L

  P = int(np.lcm(L, D)) // L
  iq_step = P * L // D

  mesh = plsc.VectorSubcoreMesh(core_axis_name="c", subcore_axis_name="s")

  @functools.partial(
      pl.kernel,
      mesh=mesh,
      compiler_params=pltpu.CompilerParams(
          use_tc_tiling_on_sc=False, needs_layout_passes=False
      ),
      out_type=jax.ShapeDtypeStruct((B * D,), jnp.float32),
      scratch_types=[
          pltpu.VMEM((b_per_w,), jnp.int32),
          pltpu.VMEM((words_per_w,), jnp.int32),
          pltpu.VMEM((words_per_w,), jnp.float32),
          pltpu.SemaphoreType.DMA,
      ],
  )
  def gather_kernel(ids_hbm, data_hbm, out_hbm, ids_v, widx_v, out_v, sem):
    wid = lax.axis_index("s") * NC + lax.axis_index("c")
    pltpu.sync_copy(ids_hbm.at[pl.ds(wid * b_per_w, b_per_w)], ids_v)
    lane = lax.iota(jnp.int32, L)
    iq_pat = []
    rqv_pat = []
    for r in range(P):
      q = r * L + lane
      iq_r = lax.div(q, D)
      iq_pat.append(iq_r)
      rqv_pat.append((q - iq_r * D) * V)
    copies = []
    for j in range(idx_rows):
      for u in range(vecs_per_row):
        t = j * vecs_per_row + u
        s, r = divmod(t, P)
        idv = plsc.load_gather(ids_v, [iq_pat[r] + (s * iq_step)])
        widx_v[pl.ds(t * L, L)] = idv + rqv_pat[r]
      copies.append(
          pltpu.async_copy(
              data_hbm.at[widx_v.at[pl.ds(j * _CHUNK, _CHUNK)]],
              out_v.at[pl.ds(j * _CHUNK, _CHUNK)],
              sem,
          )
      )
    for cp in copies:
      cp.wait()
    pltpu.sync_copy(out_v, out_hbm.at[pl.ds(wid * words_per_w, words_per_w)])

  return gather_kernel


def kernel(ids, data):
  B, = ids.shape
  V, D = data.shape
  flat = _gather_call(B, V, D)(ids, data.T.reshape(V * D))
  return flat.reshape(B, D)


# speedup vs baseline: 4.3623x; 1.5529x over previous
import functools

import jax
import jax.numpy as jnp
from jax import lax
from jax.experimental import pallas as pl
from jax.experimental.pallas import tpu as pltpu
from jax.experimental.pallas import tpu_sc as plsc

_CHUNK = 128


def _gather_call(B, V, D):
  info = plsc.get_sparse_core_info()
  NC, NS, L = info.num_cores, info.num_subcores, info.num_lanes
  NW = NC * NS
  b_per_w = B // NW
  words_per_w = b_per_w * D
  idx_rows = words_per_w // _CHUNK
  vecs_per_row = _CHUNK // L
  assert B % (NW * _CHUNK) == 0 and b_per_w % _CHUNK == 0

  mesh = plsc.VectorSubcoreMesh(core_axis_name="c", subcore_axis_name="s")

  @functools.partial(
      pl.kernel,
      mesh=mesh,
      compiler_params=pltpu.CompilerParams(
          use_tc_tiling_on_sc=False, needs_layout_passes=False
      ),
      out_type=jax.ShapeDtypeStruct((B * D,), jnp.float32),
      scratch_types=[
          pltpu.VMEM((b_per_w,), jnp.int32),
          pltpu.VMEM((words_per_w,), jnp.int32),
          pltpu.VMEM((words_per_w,), jnp.float32),
          pltpu.SemaphoreType.DMA,
          pltpu.SemaphoreType.DMA,
      ],
  )
  def gather_kernel(ids_hbm, data_hbm, out_hbm, ids_v, widx_v, out_v, sem,
                    out_sem):
    wid = lax.axis_index("s") * NC + lax.axis_index("c")
    pltpu.sync_copy(ids_hbm.at[pl.ds(wid * b_per_w, b_per_w)], ids_v)
    copies = []
    for j in range(idx_rows):
      jj = j * _CHUNK // b_per_w
      for u in range(vecs_per_row):
        p = j * _CHUNK + u * L
        widx_v[pl.ds(p, L)] = ids_v[pl.ds(p % b_per_w, L)] + (jj * V)
      copies.append(
          pltpu.async_copy(
              data_hbm.at[widx_v.at[pl.ds(j * _CHUNK, _CHUNK)]],
              out_v.at[pl.ds(j * _CHUNK, _CHUNK)],
              sem,
          )
      )
    for cp in copies:
      cp.wait()
    outs = [
        pltpu.async_copy(
            out_v.at[pl.ds(jj * b_per_w, b_per_w)],
            out_hbm.at[pl.ds(jj * B + wid * b_per_w, b_per_w)],
            out_sem,
        )
        for jj in range(D)
    ]
    for cp in outs:
      cp.wait()

  return gather_kernel


def kernel(ids, data):
  B, = ids.shape
  V, D = data.shape
  flat = _gather_call(B, V, D)(ids, data.T.reshape(V * D))
  return flat.reshape(D, B).T


# speedup vs baseline: 4.4097x; 1.0109x over previous
import functools

import jax
import jax.numpy as jnp
from jax import lax
from jax.experimental import pallas as pl
from jax.experimental.pallas import tpu as pltpu
from jax.experimental.pallas import tpu_sc as plsc

_CHUNK = 128


def _gather_call(B, V, D):
  info = plsc.get_sparse_core_info()
  NC, NS, L = info.num_cores, info.num_subcores, info.num_lanes
  NW = NC * NS
  b_per_w = B // NW
  words_per_w = b_per_w * D
  idx_rows = words_per_w // _CHUNK
  vecs_per_row = _CHUNK // L
  assert B % (NW * _CHUNK) == 0 and b_per_w % _CHUNK == 0

  mesh = plsc.VectorSubcoreMesh(core_axis_name="c", subcore_axis_name="s")

  @functools.partial(
      pl.kernel,
      mesh=mesh,
      compiler_params=pltpu.CompilerParams(
          use_tc_tiling_on_sc=False, needs_layout_passes=False
      ),
      out_type=jax.ShapeDtypeStruct((B * D,), jnp.float32),
      scratch_types=[
          pltpu.VMEM((b_per_w,), jnp.int32),
          pltpu.VMEM((words_per_w,), jnp.int32),
          pltpu.VMEM((words_per_w,), jnp.float32),
          [pltpu.SemaphoreType.DMA] * D,
          pltpu.SemaphoreType.DMA,
      ],
  )
  def gather_kernel(ids_hbm, data_hbm, out_hbm, ids_v, widx_v, out_v, sems,
                    out_sem):
    wid = lax.axis_index("s") * NC + lax.axis_index("c")
    pltpu.sync_copy(ids_hbm.at[pl.ds(wid * b_per_w, b_per_w)], ids_v)
    copies = []
    for j in range(idx_rows):
      jj = j * _CHUNK // b_per_w
      for u in range(vecs_per_row):
        p = j * _CHUNK + u * L
        widx_v[pl.ds(p, L)] = ids_v[pl.ds(p % b_per_w, L)] + (jj * V)
      copies.append(
          pltpu.async_copy(
              data_hbm.at[widx_v.at[pl.ds(j * _CHUNK, _CHUNK)]],
              out_v.at[pl.ds(j * _CHUNK, _CHUNK)],
              sems[jj],
          )
      )
    chunks_per_plane = b_per_w // _CHUNK
    outs = []
    for jj in range(D):
      for k in range(chunks_per_plane):
        copies[jj * chunks_per_plane + k].wait()
      outs.append(
          pltpu.async_copy(
              out_v.at[pl.ds(jj * b_per_w, b_per_w)],
              out_hbm.at[pl.ds(jj * B + wid * b_per_w, b_per_w)],
              out_sem,
          )
      )
    for cp in outs:
      cp.wait()

  return gather_kernel


def kernel(ids, data):
  B, = ids.shape
  V, D = data.shape
  flat = _gather_call(B, V, D)(ids, data.T.reshape(V * D))
  return flat.reshape(D, B).T


# speedup vs baseline: 4.4282x; 1.0042x over previous
import functools

import jax
import jax.numpy as jnp
from jax import lax
from jax.experimental import pallas as pl
from jax.experimental.pallas import tpu as pltpu
from jax.experimental.pallas import tpu_sc as plsc

_CHUNK = 128


def _gather_call(B, V, D):
  info = plsc.get_sparse_core_info()
  NC, NS, L = info.num_cores, info.num_subcores, info.num_lanes
  NW = NC * NS
  b_per_w = B // NW
  words_per_w = b_per_w * D
  idx_rows = words_per_w // _CHUNK
  vecs_per_row = _CHUNK // L
  assert B % (NW * _CHUNK) == 0 and b_per_w % _CHUNK == 0

  mesh = plsc.VectorSubcoreMesh(core_axis_name="c", subcore_axis_name="s")

  @functools.partial(
      pl.kernel,
      mesh=mesh,
      compiler_params=pltpu.CompilerParams(
          use_tc_tiling_on_sc=False, needs_layout_passes=False
      ),
      out_type=jax.ShapeDtypeStruct((B * D,), jnp.float32),
      scratch_types=[
          pltpu.VMEM((words_per_w,), jnp.int32),
          pltpu.VMEM((words_per_w,), jnp.float32),
          [pltpu.SemaphoreType.DMA] * D,
          pltpu.SemaphoreType.DMA,
      ],
  )
  def gather_kernel(ids_hbm, data_hbm, out_hbm, widx_v, out_v, sems,
                    out_sem):
    wid = lax.axis_index("s") * NC + lax.axis_index("c")
    pltpu.sync_copy(ids_hbm.at[pl.ds(wid * b_per_w, b_per_w)],
                    widx_v.at[pl.ds(0, b_per_w)])
    copies = []
    for j in range(idx_rows):
      jj = j * _CHUNK // b_per_w
      if jj > 0:
        for u in range(vecs_per_row):
          p = j * _CHUNK + u * L
          widx_v[pl.ds(p, L)] = widx_v[pl.ds(p % b_per_w, L)] + (jj * V)
      copies.append(
          pltpu.async_copy(
              data_hbm.at[widx_v.at[pl.ds(j * _CHUNK, _CHUNK)]],
              out_v.at[pl.ds(j * _CHUNK, _CHUNK)],
              sems[jj],
          )
      )
    chunks_per_plane = b_per_w // _CHUNK
    outs = []
    for jj in range(D):
      for k in range(chunks_per_plane):
        copies[jj * chunks_per_plane + k].wait()
      outs.append(
          pltpu.async_copy(
              out_v.at[pl.ds(jj * b_per_w, b_per_w)],
              out_hbm.at[pl.ds(jj * B + wid * b_per_w, b_per_w)],
              out_sem,
          )
      )
    for cp in outs:
      cp.wait()

  return gather_kernel


def kernel(ids, data):
  B, = ids.shape
  V, D = data.shape
  flat = _gather_call(B, V, D)(ids, data.T.reshape(V * D))
  return flat.reshape(D, B).T


# speedup vs baseline: 4.7616x; 1.0753x over previous
import functools

import jax
import jax.numpy as jnp
from jax import lax
from jax.experimental import pallas as pl
from jax.experimental.pallas import tpu as pltpu
from jax.experimental.pallas import tpu_sc as plsc

_CHUNK = 128
_SUB = 8


def _gather_call(B, V, D):
  info = plsc.get_sparse_core_info()
  NC, NS, L = info.num_cores, info.num_subcores, info.num_lanes
  NW = NC * NS
  b_per_w = B // NW
  c_per_w = b_per_w // _CHUNK
  vecs = _CHUNK // L
  assert B % (NW * _CHUNK) == 0 and D <= _SUB

  mesh = plsc.VectorSubcoreMesh(core_axis_name="c", subcore_axis_name="s")

  @functools.partial(
      pl.kernel,
      mesh=mesh,
      compiler_params=pltpu.CompilerParams(
          use_tc_tiling_on_sc=False, needs_layout_passes=False
      ),
      out_type=jax.ShapeDtypeStruct((B // _CHUNK, _SUB, _CHUNK), jnp.float32),
      scratch_types=[
          pltpu.VMEM((b_per_w,), jnp.int32),
          pltpu.VMEM((c_per_w, D, _CHUNK), jnp.int32),
          pltpu.VMEM((c_per_w, _SUB, _CHUNK), jnp.float32),
          [pltpu.SemaphoreType.DMA] * 4,
          pltpu.SemaphoreType.DMA,
      ],
  )
  def gather_kernel(ids_hbm, data_hbm, out_hbm, ids_v, widx_v, out_v, sems,
                    out_sem):
    wid = lax.axis_index("s") * NC + lax.axis_index("c")
    pltpu.sync_copy(ids_hbm.at[pl.ds(wid * b_per_w, b_per_w)], ids_v)
    copies = []
    for c in range(c_per_w):
      for j in range(D):
        for u in range(vecs):
          idv = ids_v[pl.ds(c * _CHUNK + u * L, L)]
          widx_v[c, j, pl.ds(u * L, L)] = idv + (j * V) if j else idv
        copies.append(
            pltpu.async_copy(
                data_hbm.at[widx_v.at[c, j]],
                out_v.at[c, j],
                sems[c],
            )
        )
    outs = []
    for c in range(c_per_w):
      for k in range(D):
        copies[c * D + k].wait()
      outs.append(
          pltpu.async_copy(
              out_v.at[c],
              out_hbm.at[wid * c_per_w + c],
              out_sem,
          )
      )
    for cp in outs:
      cp.wait()

  return gather_kernel


def kernel(ids, data):
  B, = ids.shape
  V, D = data.shape
  img = _gather_call(B, V, D)(ids, data.T.reshape(V * D))
  return img.transpose(0, 2, 1).reshape(B, _SUB)[:, :D]
